# baseline (speedup 1.0000x reference)
import functools

import jax
import jax.numpy as jnp
from jax.experimental import pallas as pl
from jax.experimental.pallas import tpu as pltpu
from jax.experimental.pallas import tpu_sc as plsc

N_NODES = 1024
N_NEIGH = 32
T = 8
B = 16
F_IN = 128
EMB = 128
HID = 128
NCLS = 10

INV_DEG = 1.0 / (N_NEIGH + 1)
BT = 8
NG = B // BT

_SC_CORES = 2
_SC_SUBCORES = 16
_NW = _SC_CORES * _SC_SUBCORES
_RPW = N_NODES // _NW


def _sc_build_a(adj_hbm, a_hbm, adj_v, row_v):
    wid = jax.lax.axis_index("s") * _SC_CORES + jax.lax.axis_index("c")
    base = wid * _RPW
    pltpu.sync_copy(adj_hbm.at[pl.ds(base, _RPW)], adj_v)
    ones = jnp.full((16,), 1.0, jnp.float32)
    zeros = jnp.zeros((16,), jnp.float32)
    lane0 = jax.lax.iota(jnp.int32, 16) == 0
    full = jnp.ones((16,), jnp.bool_)

    def row_body(r, carry):
        for k in range(N_NODES // 16):
            row_v[pl.ds(k * 16, 16)] = zeros
        for j in range(N_NEIGH // 16):
            idx = adj_v[r, pl.ds(j * 16, 16)]
            plsc.addupdate_scatter(row_v, [idx], ones, mask=full)
        self_idx = jnp.full((16,), base + r, jnp.int32)
        plsc.addupdate_scatter(row_v, [self_idx], ones, mask=lane0)
        pltpu.sync_copy(row_v, a_hbm.at[base + r])
        return carry

    jax.lax.fori_loop(0, _RPW, row_body, 0)


def _tc_body(a_ref, x_ref, w1_ref, b1_ref, w2_ref, b2_ref,
             wih_ref, whh_ref, bg_ref, wc_ref, bc_ref,
             out_ref, a_scr, seq_scr):
    t = pl.program_id(0)
    g = pl.program_id(1)

    @pl.when((t == 0) & (g == 0))
    def _cast_a():
        a_scr[...] = a_ref[...].astype(jnp.bfloat16)

    a = a_scr[...]
    w1 = w1_ref[...].astype(jnp.bfloat16)
    w2 = w2_ref[...].astype(jnp.bfloat16)
    b1t = jnp.concatenate([b1_ref[...]] * BT, axis=1)
    b2t = jnp.concatenate([b2_ref[...]] * BT, axis=1)
    y = jnp.concatenate(
        [jnp.dot(x_ref[0, 0, b].astype(jnp.bfloat16), w1,
                 preferred_element_type=jnp.float32).astype(jnp.bfloat16)
         for b in range(BT)],
        axis=1)
    z = jnp.dot(a, y, preferred_element_type=jnp.float32)
    h1 = (jnp.maximum(z * INV_DEG + b1t, 0.0)).astype(jnp.bfloat16)
    y2 = jnp.concatenate(
        [jnp.dot(h1[:, b * F_IN:(b + 1) * F_IN], w2,
                 preferred_element_type=jnp.float32).astype(jnp.bfloat16)
         for b in range(BT)],
        axis=1)
    z2 = jnp.dot(a, y2, preferred_element_type=jnp.float32)
    h2 = jnp.maximum(z2 * INV_DEG + b2t, 0.0)
    s = jnp.sum(h2, axis=0)
    seq_scr[t, pl.ds(g * BT, BT), :] = s.reshape(BT, EMB)

    @pl.when((t == T - 1) & (g == NG - 1))
    def _lstm_cls():
        bg = bg_ref[...]
        h = jnp.zeros((B, HID), dtype=jnp.float32)
        c = jnp.zeros((B, HID), dtype=jnp.float32)
        for tt in range(T):
            x = seq_scr[tt]
            gates = (jax.lax.dot_general(x, wih_ref[...], (((1,), (1,)), ((), ())),
                                         preferred_element_type=jnp.float32)
                     + jax.lax.dot_general(h, whh_ref[...], (((1,), (1,)), ((), ())),
                                           preferred_element_type=jnp.float32)
                     + bg)
            i = jax.nn.sigmoid(gates[:, 0 * HID:1 * HID])
            f = jax.nn.sigmoid(gates[:, 1 * HID:2 * HID])
            gg = jnp.tanh(gates[:, 2 * HID:3 * HID])
            o = jax.nn.sigmoid(gates[:, 3 * HID:4 * HID])
            c = f * c + i * gg
            h = o * jnp.tanh(c)
        hr = jnp.maximum(h, 0.0)
        logits = jax.lax.dot_general(hr, wc_ref[...], (((1,), (1,)), ((), ())),
                                     preferred_element_type=jnp.float32) + bc_ref[...]
        logits = logits - jnp.max(logits, axis=1, keepdims=True)
        e = jnp.exp(logits)
        out_ref[...] = e / jnp.sum(e, axis=1, keepdims=True)


@jax.jit
def kernel(node_features, adjacent_mappings, W1, b1, W2, b2, W_ih, W_hh, b_ih, b_hh, Wc, bc):
    adj = adjacent_mappings.astype(jnp.int32)
    nf = node_features.reshape(T, NG, BT, N_NODES, F_IN)

    a_mat = pl.kernel(
        _sc_build_a,
        out_type=jax.ShapeDtypeStruct((N_NODES, N_NODES), jnp.float32),
        mesh=plsc.VectorSubcoreMesh(core_axis_name="c", subcore_axis_name="s"),
        scratch_types=[pltpu.VMEM((_RPW, N_NEIGH), jnp.int32),
                       pltpu.VMEM((N_NODES,), jnp.float32)],
        compiler_params=pltpu.CompilerParams(needs_layout_passes=False),
    )(adj)

    out = pl.pallas_call(
        _tc_body,
        grid=(T, NG),
        in_specs=[
            pl.BlockSpec((N_NODES, N_NODES), lambda t, g: (0, 0)),
            pl.BlockSpec((1, 1, BT, N_NODES, F_IN), lambda t, g: (t, g, 0, 0, 0)),
            pl.BlockSpec((F_IN, F_IN), lambda t, g: (0, 0)),
            pl.BlockSpec((1, F_IN), lambda t, g: (0, 0)),
            pl.BlockSpec((F_IN, EMB), lambda t, g: (0, 0)),
            pl.BlockSpec((1, EMB), lambda t, g: (0, 0)),
            pl.BlockSpec((4 * HID, EMB), lambda t, g: (0, 0)),
            pl.BlockSpec((4 * HID, HID), lambda t, g: (0, 0)),
            pl.BlockSpec((1, 4 * HID), lambda t, g: (0, 0)),
            pl.BlockSpec((NCLS, HID), lambda t, g: (0, 0)),
            pl.BlockSpec((1, NCLS), lambda t, g: (0, 0)),
        ],
        out_specs=pl.BlockSpec((B, NCLS), lambda t, g: (0, 0)),
        out_shape=jax.ShapeDtypeStruct((B, NCLS), jnp.float32),
        scratch_shapes=[pltpu.VMEM((N_NODES, N_NODES), jnp.bfloat16),
                        pltpu.VMEM((T, B, EMB), jnp.float32)],
    )(a_mat, nf, W1, b1.reshape(1, F_IN), W2, b2.reshape(1, EMB),
      W_ih, W_hh, (b_ih + b_hh).reshape(1, 4 * HID), Wc, bc.reshape(1, NCLS))

    return out


# speedup vs baseline: 1.0140x; 1.0140x over previous
import functools

import jax
import jax.numpy as jnp
from jax.experimental import pallas as pl
from jax.experimental.pallas import tpu as pltpu
from jax.experimental.pallas import tpu_sc as plsc

N_NODES = 1024
N_NEIGH = 32
T = 8
B = 16
F_IN = 128
EMB = 128
HID = 128
NCLS = 10

INV_DEG = 1.0 / (N_NEIGH + 1)
BT = 8
NG = B // BT

_SC_CORES = 2
_SC_SUBCORES = 16
_NW = _SC_CORES * _SC_SUBCORES
_RPW = N_NODES // _NW


def _sc_build_a(adj_hbm, a_hbm, adj_v, rows_v):
    wid = jax.lax.axis_index("s") * _SC_CORES + jax.lax.axis_index("c")
    base = wid * _RPW
    pltpu.sync_copy(adj_hbm.at[pl.ds(base, _RPW)], adj_v)
    ones = jnp.full((16,), 1.0, jnp.float32)
    zeros = jnp.zeros((16,), jnp.float32)
    lane0 = jax.lax.iota(jnp.int32, 16) == 0
    full = jnp.ones((16,), jnp.bool_)

    def row_body(r, carry):
        for k in range(N_NODES // 16):
            rows_v[r, pl.ds(k * 16, 16)] = zeros
        return carry

    jax.lax.fori_loop(0, _RPW, row_body, 0)

    def scat_body(r, carry):
        row_idx = jnp.full((16,), r, jnp.int32)
        for j in range(N_NEIGH // 16):
            idx = adj_v[r, pl.ds(j * 16, 16)]
            plsc.addupdate_scatter(rows_v, [row_idx, idx], ones, mask=full)
        self_idx = jnp.full((16,), base + r, jnp.int32)
        plsc.addupdate_scatter(rows_v, [row_idx, self_idx], ones, mask=lane0)
        return carry

    jax.lax.fori_loop(0, _RPW, scat_body, 0)
    pltpu.sync_copy(rows_v, a_hbm.at[pl.ds(base, _RPW)])


def _tc_body(a_ref, x_ref, w1_ref, b1_ref, w2_ref, b2_ref,
             wih_ref, whh_ref, bg_ref, wc_ref, bc_ref,
             out_ref, a_scr, seq_scr):
    t = pl.program_id(0)
    g = pl.program_id(1)

    @pl.when((t == 0) & (g == 0))
    def _cast_a():
        a_scr[...] = a_ref[...].astype(jnp.bfloat16)

    a = a_scr[...]
    w1 = w1_ref[...].astype(jnp.bfloat16)
    w2 = w2_ref[...].astype(jnp.bfloat16)
    b1t = jnp.concatenate([b1_ref[...]] * BT, axis=1)
    b2t = jnp.concatenate([b2_ref[...]] * BT, axis=1)
    y = jnp.concatenate(
        [jnp.dot(x_ref[0, 0, b].astype(jnp.bfloat16), w1,
                 preferred_element_type=jnp.float32).astype(jnp.bfloat16)
         for b in range(BT)],
        axis=1)
    z = jnp.dot(a, y, preferred_element_type=jnp.float32)
    h1 = (jnp.maximum(z * INV_DEG + b1t, 0.0)).astype(jnp.bfloat16)
    y2 = jnp.concatenate(
        [jnp.dot(h1[:, b * F_IN:(b + 1) * F_IN], w2,
                 preferred_element_type=jnp.float32).astype(jnp.bfloat16)
         for b in range(BT)],
        axis=1)
    z2 = jnp.dot(a, y2, preferred_element_type=jnp.float32)
    h2 = jnp.maximum(z2 * INV_DEG + b2t, 0.0)
    s = jnp.sum(h2, axis=0)
    seq_scr[t, pl.ds(g * BT, BT), :] = s.reshape(BT, EMB)

    @pl.when((t == T - 1) & (g == NG - 1))
    def _lstm_cls():
        bg = bg_ref[...]
        h = jnp.zeros((B, HID), dtype=jnp.float32)
        c = jnp.zeros((B, HID), dtype=jnp.float32)
        for tt in range(T):
            x = seq_scr[tt]
            gates = (jax.lax.dot_general(x, wih_ref[...], (((1,), (1,)), ((), ())),
                                         preferred_element_type=jnp.float32)
                     + jax.lax.dot_general(h, whh_ref[...], (((1,), (1,)), ((), ())),
                                           preferred_element_type=jnp.float32)
                     + bg)
            i = jax.nn.sigmoid(gates[:, 0 * HID:1 * HID])
            f = jax.nn.sigmoid(gates[:, 1 * HID:2 * HID])
            gg = jnp.tanh(gates[:, 2 * HID:3 * HID])
            o = jax.nn.sigmoid(gates[:, 3 * HID:4 * HID])
            c = f * c + i * gg
            h = o * jnp.tanh(c)
        hr = jnp.maximum(h, 0.0)
        logits = jax.lax.dot_general(hr, wc_ref[...], (((1,), (1,)), ((), ())),
                                     preferred_element_type=jnp.float32) + bc_ref[...]
        logits = logits - jnp.max(logits, axis=1, keepdims=True)
        e = jnp.exp(logits)
        out_ref[...] = e / jnp.sum(e, axis=1, keepdims=True)


@jax.jit
def kernel(node_features, adjacent_mappings, W1, b1, W2, b2, W_ih, W_hh, b_ih, b_hh, Wc, bc):
    adj = adjacent_mappings.astype(jnp.int32)
    nf = node_features.reshape(T, NG, BT, N_NODES, F_IN)

    a_mat = pl.kernel(
        _sc_build_a,
        out_type=jax.ShapeDtypeStruct((N_NODES, N_NODES), jnp.float32),
        mesh=plsc.VectorSubcoreMesh(core_axis_name="c", subcore_axis_name="s"),
        scratch_types=[pltpu.VMEM((_RPW, N_NEIGH), jnp.int32),
                       pltpu.VMEM((_RPW, N_NODES), jnp.float32)],
        compiler_params=pltpu.CompilerParams(needs_layout_passes=False),
    )(adj)

    out = pl.pallas_call(
        _tc_body,
        grid=(T, NG),
        in_specs=[
            pl.BlockSpec((N_NODES, N_NODES), lambda t, g: (0, 0)),
            pl.BlockSpec((1, 1, BT, N_NODES, F_IN), lambda t, g: (t, g, 0, 0, 0)),
            pl.BlockSpec((F_IN, F_IN), lambda t, g: (0, 0)),
            pl.BlockSpec((1, F_IN), lambda t, g: (0, 0)),
            pl.BlockSpec((F_IN, EMB), lambda t, g: (0, 0)),
            pl.BlockSpec((1, EMB), lambda t, g: (0, 0)),
            pl.BlockSpec((4 * HID, EMB), lambda t, g: (0, 0)),
            pl.BlockSpec((4 * HID, HID), lambda t, g: (0, 0)),
            pl.BlockSpec((1, 4 * HID), lambda t, g: (0, 0)),
            pl.BlockSpec((NCLS, HID), lambda t, g: (0, 0)),
            pl.BlockSpec((1, NCLS), lambda t, g: (0, 0)),
        ],
        out_specs=pl.BlockSpec((B, NCLS), lambda t, g: (0, 0)),
        out_shape=jax.ShapeDtypeStruct((B, NCLS), jnp.float32),
        scratch_shapes=[pltpu.VMEM((N_NODES, N_NODES), jnp.bfloat16),
                        pltpu.VMEM((T, B, EMB), jnp.float32)],
    )(a_mat, nf, W1, b1.reshape(1, F_IN), W2, b2.reshape(1, EMB),
      W_ih, W_hh, (b_ih + b_hh).reshape(1, 4 * HID), Wc, bc.reshape(1, NCLS))

    return out
